# baseline (speedup 1.0000x reference)
import dataclasses
import functools

import jax
import jax.numpy as jnp
from jax import lax
from jax.experimental import pallas as pl
from jax.experimental.pallas import tpu as pltpu
from jax.experimental.pallas import tpu_sc as plsc

N = 10000
E = 160000
D = 256
H = 128

_NSUB = 16
_CH = 112
_NCH = 90
_E_PAD = _NSUB * _NCH * _CH
_RZ = 624

_RB = 400


def _sc_body(attr_row, x_hbm, edata_hbm, out_hbm,
             e0, e1, e2, e3, r0, r1, r2,
             es0, es1, es2, es3, gs0, gs1, gs2, ss0, ss1, ss2, acc_sh):
    c = lax.axis_index("c")
    sid = lax.axis_index("s")
    ebufs = (e0, e1, e2, e3)
    esem = (es0, es1, es2, es3)
    rbufs = (r0, r1, r2)
    gsem = (gs0, gs1, gs2)
    ssem = (ss0, ss1, ss2)
    zero16 = jnp.zeros((16,), jnp.float32)

    @pl.loop(0, _CH)
    def _zrow(i):
        for g in range(0, H, 16):
            r0[i, pl.ds(g, 16)] = zero16

    rbase = sid * _RZ

    @pl.loop(0, 5)
    def _zacc(t):
        pltpu.sync_copy(r0, acc_sh.at[pl.ds(rbase + t * _CH, _CH)])

    pltpu.sync_copy(r0.at[pl.ds(0, 64)],
                    acc_sh.at[pl.ds(rbase + 5 * _CH, 64)])

    @pl.when(sid == _NSUB - 1)
    def _ztail():
        pltpu.sync_copy(r0.at[pl.ds(0, 16)],
                        acc_sh.at[pl.ds(_NSUB * _RZ, 16)])

    cbase = c * N
    csplat = jnp.full((16,), cbase, jnp.int32)

    def issue_idx(jj, eb):
        pltpu.async_copy(edata_hbm.at[sid, jj], ebufs[eb], esem[eb])

    def idx_wait(jj, eb):
        pltpu.make_async_copy(edata_hbm.at[sid, jj], ebufs[eb],
                              esem[eb]).wait()

    def adjust(eb):
        e = ebufs[eb]
        for g in range(0, _CH, 16):
            e[0, pl.ds(g, 16)] = e[0, pl.ds(g, 16)] + csplat

    def issue_gather(rb, eb):
        pltpu.async_copy(x_hbm.at[ebufs[eb].at[0]], rbufs[rb], gsem[rb])

    def gather_wait(rb, eb):
        pltpu.make_async_copy(x_hbm.at[ebufs[eb].at[0]], rbufs[rb],
                              gsem[rb]).wait()

    def issue_scatter(rb, eb):
        pltpu.async_copy(rbufs[rb], acc_sh.at[ebufs[eb].at[1]], ssem[rb],
                         add=True)

    def scatter_wait(rb, eb):
        pltpu.make_async_copy(rbufs[rb], acc_sh.at[ebufs[eb].at[1]],
                              ssem[rb]).wait()

    def scale(rb, eb):
        buf = rbufs[rb]
        att = ebufs[eb]
        two = jnp.full((16,), attr_row, jnp.int32)

        @pl.loop(0, _CH, step=2)
        def _(i):
            for u in range(2):
                w = plsc.bitcast(
                    plsc.load_gather(
                        att, [two, jnp.full((16,), i + u, jnp.int32)]),
                    jnp.float32)
                for g in range(0, H, 16):
                    buf[i + u, pl.ds(g, 16)] = buf[i + u, pl.ds(g, 16)] * w

    def body(k, first=False):
        r = k % 3
        eb = k % 4
        gather_wait(r, eb)
        if k + 1 < _NCH:
            idx_wait(k + 1, (k + 1) % 4)
            adjust((k + 1) % 4)
        if not first:
            scatter_wait((k - 2) % 3, (k - 2) % 4)
        if k + 1 < _NCH:
            issue_gather((k + 1) % 3, (k + 1) % 4)
        if k + 2 < _NCH:
            issue_idx(k + 2, (k + 2) % 4)
        scale(r, eb)
        issue_scatter(r, eb)

    issue_idx(0, 0)
    issue_idx(1, 1)
    idx_wait(0, 0)
    adjust(0)
    issue_gather(0, 0)
    body(0, first=True)
    body(1, first=True)

    @pl.loop(2, 86, step=12)
    def _steady(j):
        for b12 in range(12):
            k = j + b12
            r = (2 + b12) % 3
            eb = (2 + b12) % 4
            gather_wait(r, eb)
            idx_wait(k + 1, (eb + 1) % 4)
            adjust((eb + 1) % 4)
            scatter_wait((r + 1) % 3, (eb + 2) % 4)
            issue_gather((r + 1) % 3, (eb + 1) % 4)
            issue_idx(k + 2, (eb + 2) % 4)
            scale(r, eb)
            issue_scatter(r, eb)

    for k in range(86, _NCH):
        body(k)
    scatter_wait((_NCH - 2) % 3, (_NCH - 2) % 4)
    scatter_wait((_NCH - 1) % 3, (_NCH - 1) % 4)

    plsc.subcore_barrier()

    pltpu.sync_copy(acc_sh.at[pl.ds(rbase, _RZ)],
                    out_hbm.at[pl.ds(cbase + rbase, _RZ)])

    @pl.when(sid == _NSUB - 1)
    def _wtail():
        pltpu.sync_copy(acc_sh.at[pl.ds(_NSUB * _RZ, 16)],
                        out_hbm.at[pl.ds(cbase + _NSUB * _RZ, 16)])


def _sc_layer(xcat, edata, attr_row):
    mesh = plsc.VectorSubcoreMesh(core_axis_name="c", subcore_axis_name="s")
    cp = pltpu.CompilerParams()
    if "needs_layout_passes" in pltpu.CompilerParams.__dataclass_fields__:
        cp = dataclasses.replace(cp, needs_layout_passes=False)
    kfn = pl.kernel(
        functools.partial(_sc_body, attr_row),
        out_type=jax.ShapeDtypeStruct((2 * N, H), jnp.float32),
        mesh=mesh,
        scratch_types=(
            [pltpu.VMEM((4, _CH), jnp.int32)] * 4
            + [pltpu.VMEM((_CH, H), jnp.float32)] * 3
            + [pltpu.SemaphoreType.DMA] * 10
            + [pltpu.VMEM_SHARED((N, H), jnp.float32)]
        ),
        compiler_params=cp,
    )
    return kfn(xcat, edata)


def _tcp_body(a_ref, b_ref, w_ref, bias_ref, o_ref):
    p = lax.Precision.HIGHEST
    acc = jnp.dot(a_ref[...], w_ref[0:H, :], precision=p)
    acc = acc + jnp.dot(b_ref[...], w_ref[H:D, :], precision=p)
    o_ref[...] = acc + bias_ref[...]


def _tc_pre(a, b, WT, bias):
    return pl.pallas_call(
        _tcp_body,
        grid=(N // _RB,),
        in_specs=[
            pl.BlockSpec((_RB, H), lambda i: (i, 0)),
            pl.BlockSpec((_RB, H), lambda i: (i, 0)),
            pl.BlockSpec((D, D), lambda i: (0, 0)),
            pl.BlockSpec((1, D), lambda i: (0, 0)),
        ],
        out_specs=pl.BlockSpec((_RB, D), lambda i: (i, 0)),
        out_shape=jax.ShapeDtypeStruct((N, D), jnp.float32),
    )(a, b, WT, bias)


def _tc1_body(p_ref, al_ref, ar_ref, w_ref, o_ref):
    p = lax.Precision.HIGHEST
    acc = jnp.dot(al_ref[...], w_ref[0:H, :], precision=p)
    acc = acc + jnp.dot(ar_ref[...], w_ref[H:D, :], precision=p)
    o_ref[0] = jnp.maximum(acc + p_ref[...], 0.0)


def _tc_fin1(p1, aggL, aggR, W1aT):
    return pl.pallas_call(
        _tc1_body,
        grid=(2, N // _RB),
        in_specs=[
            pl.BlockSpec((_RB, H), lambda j, i: (i, j)),
            pl.BlockSpec((_RB, H), lambda j, i: (i, 0)),
            pl.BlockSpec((_RB, H), lambda j, i: (i, 0)),
            pl.BlockSpec((D, H), lambda j, i: (0, j)),
        ],
        out_specs=pl.BlockSpec((1, _RB, H), lambda j, i: (j, i, 0)),
        out_shape=jax.ShapeDtypeStruct((2, N, H), jnp.float32),
    )(p1, aggL, aggR, W1aT)


def _tc2_body(p_ref, al_ref, ar_ref, w_ref, o_ref):
    p = lax.Precision.HIGHEST
    acc = jnp.dot(al_ref[...], w_ref[0:H, :], precision=p)
    acc = acc + jnp.dot(ar_ref[...], w_ref[H:D, :], precision=p)
    o_ref[...] = acc + p_ref[...]


def _tc_fin2(p2, aggL, aggR, W2aT):
    return pl.pallas_call(
        _tc2_body,
        grid=(N // _RB,),
        in_specs=[
            pl.BlockSpec((_RB, D), lambda i: (i, 0)),
            pl.BlockSpec((_RB, H), lambda i: (i, 0)),
            pl.BlockSpec((_RB, H), lambda i: (i, 0)),
            pl.BlockSpec((D, D), lambda i: (0, 0)),
        ],
        out_specs=pl.BlockSpec((_RB, D), lambda i: (i, 0)),
        out_shape=jax.ShapeDtypeStruct((N, D), jnp.float32),
    )(p2, aggL, aggR, W2aT)


def kernel(x, edge_index, env_edge_attr, act_edge_attr, W1, b1, W2, b2):
    pad = _E_PAD - E
    shp = (_NSUB, _NCH, _CH)
    src = jnp.pad(edge_index[0].astype(jnp.int32), (0, pad)).reshape(shp)
    dst = jnp.pad(edge_index[1].astype(jnp.int32), (0, pad)).reshape(shp)
    env = lax.bitcast_convert_type(
        jnp.pad(env_edge_attr[:, 0], (0, pad)), jnp.int32).reshape(shp)
    act = lax.bitcast_convert_type(
        jnp.pad(act_edge_attr[:, 0], (0, pad)), jnp.int32).reshape(shp)
    edata = jnp.stack([src, dst, env, act], axis=2)

    xcat = x.reshape(N, 2, H).transpose(1, 0, 2).reshape(2 * N, H)
    W1T = W1.T
    W2T = W2.T
    b1r = b1.reshape(1, D)
    b2r = b2.reshape(1, D)

    agg1 = _sc_layer(xcat, edata, 2)
    p1 = _tc_pre(xcat[:N], xcat[N:], W1T[:D], b1r)
    hs = _tc_fin1(p1, agg1[:N], agg1[N:], W1T[D:])
    hcat = hs.reshape(2 * N, H)
    agg2 = _sc_layer(hcat, edata, 3)
    p2 = _tc_pre(hs[0], hs[1], W2T[:D], b2r)
    out = _tc_fin2(p2, agg2[:N], agg2[N:], W2T[D:])
    return out


# speedup vs baseline: 1.0381x; 1.0381x over previous
import dataclasses
import functools

import jax
import jax.numpy as jnp
from jax import lax
from jax.experimental import pallas as pl
from jax.experimental.pallas import tpu as pltpu
from jax.experimental.pallas import tpu_sc as plsc

N = 10000
E = 160000
D = 256
H = 128

_NSUB = 16
_CH = 112
_NCH = 90
_E_PAD = _NSUB * _NCH * _CH
_RZ = 624

_RB = 400


def _sc_body(attr_row, x_hbm, edata_hbm, out_hbm,
             e0, e1, e2, e3, r0, r1, r2,
             es0, es1, es2, es3, gs0, gs1, gs2, ss0, ss1, ss2, acc_sh):
    c = lax.axis_index("c")
    sid = lax.axis_index("s")
    ebufs = (e0, e1, e2, e3)
    esem = (es0, es1, es2, es3)
    rbufs = (r0, r1, r2)
    gsem = (gs0, gs1, gs2)
    ssem = (ss0, ss1, ss2)
    zero16 = jnp.zeros((16,), jnp.float32)

    @pl.loop(0, _CH)
    def _zrow(i):
        for g in range(0, H, 16):
            r0[i, pl.ds(g, 16)] = zero16

    rbase = sid * _RZ

    @pl.loop(0, 5)
    def _zacc(t):
        pltpu.sync_copy(r0, acc_sh.at[pl.ds(rbase + t * _CH, _CH)])

    pltpu.sync_copy(r0.at[pl.ds(0, 64)],
                    acc_sh.at[pl.ds(rbase + 5 * _CH, 64)])

    @pl.when(sid == _NSUB - 1)
    def _ztail():
        pltpu.sync_copy(r0.at[pl.ds(0, 16)],
                        acc_sh.at[pl.ds(_NSUB * _RZ, 16)])

    coff = pl.multiple_of(c * H, H)

    def issue_idx(jj, eb):
        pltpu.async_copy(edata_hbm.at[sid, jj], ebufs[eb], esem[eb])

    def idx_wait(jj, eb):
        pltpu.make_async_copy(edata_hbm.at[sid, jj], ebufs[eb],
                              esem[eb]).wait()

    def issue_gather(rb, eb):
        pltpu.async_copy(x_hbm.at[ebufs[eb].at[0], pl.ds(coff, H)],
                         rbufs[rb], gsem[rb])

    def gather_wait(rb, eb):
        pltpu.make_async_copy(x_hbm.at[ebufs[eb].at[0], pl.ds(coff, H)],
                              rbufs[rb], gsem[rb]).wait()

    def issue_scatter(rb, eb):
        pltpu.async_copy(rbufs[rb], acc_sh.at[ebufs[eb].at[1]], ssem[rb],
                         add=True)

    def scatter_wait(rb, eb):
        pltpu.make_async_copy(rbufs[rb], acc_sh.at[ebufs[eb].at[1]],
                              ssem[rb]).wait()

    def scale(rb, eb):
        buf = rbufs[rb]
        att = ebufs[eb]
        two = jnp.full((16,), attr_row, jnp.int32)

        @pl.loop(0, _CH, step=2)
        def _(i):
            for u in range(2):
                w = plsc.bitcast(
                    plsc.load_gather(
                        att, [two, jnp.full((16,), i + u, jnp.int32)]),
                    jnp.float32)
                for g in range(0, H, 16):
                    buf[i + u, pl.ds(g, 16)] = buf[i + u, pl.ds(g, 16)] * w

    def body(k, first=False):
        r = k % 3
        eb = k % 4
        gather_wait(r, eb)
        if k + 1 < _NCH:
            idx_wait(k + 1, (k + 1) % 4)
        if not first:
            scatter_wait((k - 2) % 3, (k - 2) % 4)
        if k + 1 < _NCH:
            issue_gather((k + 1) % 3, (k + 1) % 4)
        if k + 2 < _NCH:
            issue_idx(k + 2, (k + 2) % 4)
        scale(r, eb)
        issue_scatter(r, eb)

    issue_idx(0, 0)
    issue_idx(1, 1)
    idx_wait(0, 0)
    issue_gather(0, 0)
    body(0, first=True)
    body(1, first=True)

    @pl.loop(2, 86, step=12)
    def _steady(j):
        for b12 in range(12):
            k = j + b12
            r = (2 + b12) % 3
            eb = (2 + b12) % 4
            gather_wait(r, eb)
            idx_wait(k + 1, (eb + 1) % 4)
            scatter_wait((r + 1) % 3, (eb + 2) % 4)
            issue_gather((r + 1) % 3, (eb + 1) % 4)
            issue_idx(k + 2, (eb + 2) % 4)
            scale(r, eb)
            issue_scatter(r, eb)

    for k in range(86, _NCH):
        body(k)
    scatter_wait((_NCH - 2) % 3, (_NCH - 2) % 4)
    scatter_wait((_NCH - 1) % 3, (_NCH - 1) % 4)

    plsc.subcore_barrier()

    pltpu.sync_copy(acc_sh.at[pl.ds(rbase, _RZ)],
                    out_hbm.at[pl.ds(rbase, _RZ), pl.ds(coff, H)])

    @pl.when(sid == _NSUB - 1)
    def _wtail():
        pltpu.sync_copy(acc_sh.at[pl.ds(_NSUB * _RZ, 16)],
                        out_hbm.at[pl.ds(_NSUB * _RZ, 16), pl.ds(coff, H)])


def _sc_layer(x, edata, attr_row):
    mesh = plsc.VectorSubcoreMesh(core_axis_name="c", subcore_axis_name="s")
    cp = pltpu.CompilerParams()
    if "needs_layout_passes" in pltpu.CompilerParams.__dataclass_fields__:
        cp = dataclasses.replace(cp, needs_layout_passes=False)
    kfn = pl.kernel(
        functools.partial(_sc_body, attr_row),
        out_type=jax.ShapeDtypeStruct((N, D), jnp.float32),
        mesh=mesh,
        scratch_types=(
            [pltpu.VMEM((4, _CH), jnp.int32)] * 4
            + [pltpu.VMEM((_CH, H), jnp.float32)] * 3
            + [pltpu.SemaphoreType.DMA] * 10
            + [pltpu.VMEM_SHARED((N, H), jnp.float32)]
        ),
        compiler_params=cp,
    )
    return kfn(x, edata)


def _tcp_body(x_ref, w_ref, bias_ref, o_ref):
    o_ref[...] = jnp.dot(x_ref[...], w_ref[...],
                         precision=lax.Precision.HIGHEST) + bias_ref[...]


def _tc_pre(x, WT, bias):
    return pl.pallas_call(
        _tcp_body,
        grid=(N // _RB,),
        in_specs=[
            pl.BlockSpec((_RB, D), lambda i: (i, 0)),
            pl.BlockSpec((D, D), lambda i: (0, 0)),
            pl.BlockSpec((1, D), lambda i: (0, 0)),
        ],
        out_specs=pl.BlockSpec((_RB, D), lambda i: (i, 0)),
        out_shape=jax.ShapeDtypeStruct((N, D), jnp.float32),
    )(x, WT, bias)


def _tcf_body(relu, p_ref, a_ref, w_ref, o_ref):
    acc = p_ref[...] + jnp.dot(a_ref[...], w_ref[...],
                               precision=lax.Precision.HIGHEST)
    o_ref[...] = jnp.maximum(acc, 0.0) if relu else acc


def _tc_fin(p, agg, WaT, relu):
    return pl.pallas_call(
        functools.partial(_tcf_body, relu),
        grid=(N // _RB,),
        in_specs=[
            pl.BlockSpec((_RB, D), lambda i: (i, 0)),
            pl.BlockSpec((_RB, D), lambda i: (i, 0)),
            pl.BlockSpec((D, D), lambda i: (0, 0)),
        ],
        out_specs=pl.BlockSpec((_RB, D), lambda i: (i, 0)),
        out_shape=jax.ShapeDtypeStruct((N, D), jnp.float32),
    )(p, agg, WaT)


def kernel(x, edge_index, env_edge_attr, act_edge_attr, W1, b1, W2, b2):
    pad = _E_PAD - E
    shp = (_NSUB, _NCH, _CH)
    src = jnp.pad(edge_index[0].astype(jnp.int32), (0, pad)).reshape(shp)
    dst = jnp.pad(edge_index[1].astype(jnp.int32), (0, pad)).reshape(shp)
    env = lax.bitcast_convert_type(
        jnp.pad(env_edge_attr[:, 0], (0, pad)), jnp.int32).reshape(shp)
    act = lax.bitcast_convert_type(
        jnp.pad(act_edge_attr[:, 0], (0, pad)), jnp.int32).reshape(shp)
    edata = jnp.stack([src, dst, env, act], axis=2)

    W1T = W1.T
    W2T = W2.T
    b1r = b1.reshape(1, D)
    b2r = b2.reshape(1, D)

    agg1 = _sc_layer(x, edata, 2)
    p1 = _tc_pre(x, W1T[:D], b1r)
    h = _tc_fin(p1, agg1, W1T[D:], True)
    agg2 = _sc_layer(h, edata, 3)
    p2 = _tc_pre(h, W2T[:D], b2r)
    return _tc_fin(p2, agg2, W2T[D:], False)
